# baseline (speedup 1.0000x reference)
import jax
import jax.numpy as jnp
from jax import lax
from jax.experimental import pallas as pl
from jax.experimental.pallas import tpu as pltpu
from jax.experimental.pallas import tpu_sc as plsc

BATCH = 1024
SEQ = 50
DIM = 1024
LANES = 16
NUM_CORES = 2
NUM_SUBCORES = 16
NUM_WORKERS = NUM_CORES * NUM_SUBCORES
BPW = BATCH // NUM_WORKERS
IDX_PER_W = BPW * SEQ
CHUNK = 16
NCH = IDX_PER_W // CHUNK
NBUF = 4


def _sc_encode(x_hbm, table_hbm, out_hbm, idx_v, rows_a, rows_b, rows_c,
               rows_d, acc_v, stage_v, sem_a, sem_b, sem_c, sem_d, sem_o):
    wid = lax.axis_index("s") * NUM_CORES + lax.axis_index("c")
    base = wid * BPW
    pltpu.sync_copy(x_hbm.at[wid], idx_v)

    bufs = (rows_a, rows_b, rows_c, rows_d)
    sems = (sem_a, sem_b, sem_c, sem_d)

    for k in range(NBUF):
        pltpu.make_async_copy(
            table_hbm.at[idx_v.at[pl.ds(CHUNK * k, CHUNK)]], bufs[k],
            sems[k]).start()

    def zero_body(c, carry):
        acc_v[pl.ds(c * LANES, LANES)] = jnp.zeros((LANES,), jnp.float32)
        return carry

    lax.fori_loop(0, DIM // LANES, zero_body, 0)

    def outer_body(i, carry):
        b0, p = carry
        for j in range(NBUF):
            kk = i * NBUF + j
            buf = bufs[j]
            pltpu.make_async_copy(
                table_hbm.at[pl.ds(0, CHUNK)], buf, sems[j]).wait()

            hci = 1 + ((CHUNK - p) >> 31)
            row_masks = [
                jnp.full((LANES,), (r - p) >> 31, jnp.int32)
                for r in range(CHUNK)
            ]
            hmask = jnp.full((LANES,), -hci, jnp.int32)

            @plsc.parallel_loop(0, DIM // LANES, unroll=2, carry=b0)
            def c_body(c, carry2, _buf=buf, _masks=row_masks, _hmask=hmask):
                b0c = carry2
                off = c * LANES
                zero = jnp.zeros((LANES,), jnp.float32)
                t0 = t1 = t2 = t3 = zero
                a0 = a1c = zero
                for r in range(CHUNK):
                    x = _buf[r, pl.ds(off, LANES)]
                    if r % 4 == 0:
                        t0 = t0 + x
                    elif r % 4 == 1:
                        t1 = t1 + x
                    elif r % 4 == 2:
                        t2 = t2 + x
                    else:
                        t3 = t3 + x
                    xm = lax.bitcast_convert_type(
                        lax.bitcast_convert_type(x, jnp.int32) & _masks[r],
                        jnp.float32)
                    if r % 2 == 0:
                        a0 = a0 + xm
                    else:
                        a1c = a1c + xm
                total = (t0 + t1) + (t2 + t3)
                pre = a0 + a1c
                acc = acc_v[pl.ds(off, LANES)]
                fin = acc + pre
                stage_v[b0c, pl.ds(off, LANES)] = jnp.where(
                    fin > 0.0, 1.0, -1.0)
                b_acc = lax.bitcast_convert_type(acc, jnp.int32)
                b_np = lax.bitcast_convert_type(-pre, jnp.int32)
                blend = b_acc ^ ((b_acc ^ b_np) & _hmask)
                acc_v[pl.ds(off, LANES)] = total + lax.bitcast_convert_type(
                    blend, jnp.float32)
                return b0c

            @pl.when(hci > 0)
            def _():
                pltpu.make_async_copy(
                    stage_v.at[b0], out_hbm.at[base + b0], sem_o).start()

            @pl.when(kk + NBUF < NCH)
            def _():
                pltpu.make_async_copy(
                    table_hbm.at[idx_v.at[pl.ds(CHUNK * (kk + NBUF), CHUNK)]],
                    buf, sems[j]).start()

            b0 = b0 + hci
            p = p + SEQ * hci - CHUNK
        return b0, p

    lax.fori_loop(0, NCH // NBUF, outer_body,
                  (jnp.int32(0), jnp.int32(SEQ)))

    pltpu.make_async_copy(stage_v, out_hbm.at[pl.ds(base, BPW)],
                          sem_o).wait()


@jax.jit
def kernel(x, symbol):
    mesh = plsc.VectorSubcoreMesh(core_axis_name="c", subcore_axis_name="s")
    f = pl.kernel(
        _sc_encode,
        mesh=mesh,
        out_type=jax.ShapeDtypeStruct((BATCH, DIM), jnp.float32),
        scratch_types=[
            pltpu.VMEM((IDX_PER_W,), jnp.int32),
            pltpu.VMEM((CHUNK, DIM), jnp.float32),
            pltpu.VMEM((CHUNK, DIM), jnp.float32),
            pltpu.VMEM((CHUNK, DIM), jnp.float32),
            pltpu.VMEM((CHUNK, DIM), jnp.float32),
            pltpu.VMEM((DIM,), jnp.float32),
            pltpu.VMEM((BPW, DIM), jnp.float32),
            pltpu.SemaphoreType.DMA,
            pltpu.SemaphoreType.DMA,
            pltpu.SemaphoreType.DMA,
            pltpu.SemaphoreType.DMA,
            pltpu.SemaphoreType.DMA,
        ],
    )
    return f(x.reshape(NUM_WORKERS, IDX_PER_W), symbol)


# speedup vs baseline: 1.0355x; 1.0355x over previous
import jax
import jax.numpy as jnp
from jax import lax
from jax.experimental import pallas as pl
from jax.experimental.pallas import tpu as pltpu
from jax.experimental.pallas import tpu_sc as plsc

BATCH = 1024
SEQ = 50
DIM = 1024
LANES = 16
NUM_CORES = 2
NUM_SUBCORES = 16
NUM_WORKERS = NUM_CORES * NUM_SUBCORES
BPW = BATCH // NUM_WORKERS
IDX_PER_W = BPW * SEQ
CHUNK = 32
NCH = IDX_PER_W // CHUNK
NBUF = 3
NMAIN = NCH - 2
NSTAGE = 16


def _sc_encode(x_hbm, table_hbm, out_hbm, idx_v, rows_a, rows_b, rows_c,
               acc_v, stage_v, sem_a, sem_b, sem_c, sem_o):
    wid = lax.axis_index("s") * NUM_CORES + lax.axis_index("c")
    base = wid * BPW
    pltpu.sync_copy(x_hbm.at[wid], idx_v)

    bufs = (rows_a, rows_b, rows_c)
    sems = (sem_a, sem_b, sem_c)

    for k in range(2):
        pltpu.make_async_copy(
            table_hbm.at[idx_v.at[pl.ds(CHUNK * k, CHUNK)]], bufs[k],
            sems[k]).start()

    def zero_body(c, carry):
        acc_v[pl.ds(c * LANES, LANES)] = jnp.zeros((LANES,), jnp.float32)
        return carry

    lax.fori_loop(0, DIM // LANES, zero_body, 0)

    def do_chunk(buf, sem, b0, p):
        pltpu.make_async_copy(table_hbm.at[pl.ds(0, CHUNK)], buf, sem).wait()

        hci = 1 + ((CHUNK - p) >> 31)
        srow = b0 & (NSTAGE - 1)
        row_masks = [
            jnp.full((LANES,), (r - p) >> 31, jnp.int32)
            for r in range(CHUNK)
        ]
        hmask = jnp.full((LANES,), -hci, jnp.int32)

        @plsc.parallel_loop(0, DIM // LANES, unroll=2)
        def c_body(c, _buf=buf, _masks=row_masks, _hmask=hmask, _srow=srow):
            off = c * LANES
            zero = jnp.zeros((LANES,), jnp.float32)
            t0 = t1 = t2 = t3 = zero
            a0 = a1c = zero
            for r in range(CHUNK):
                x = _buf[r, pl.ds(off, LANES)]
                if r % 4 == 0:
                    t0 = t0 + x
                elif r % 4 == 1:
                    t1 = t1 + x
                elif r % 4 == 2:
                    t2 = t2 + x
                else:
                    t3 = t3 + x
                xm = lax.bitcast_convert_type(
                    lax.bitcast_convert_type(x, jnp.int32) & _masks[r],
                    jnp.float32)
                if r % 2 == 0:
                    a0 = a0 + xm
                else:
                    a1c = a1c + xm
            total = (t0 + t1) + (t2 + t3)
            pre = a0 + a1c
            acc = acc_v[pl.ds(off, LANES)]
            fin = acc + pre
            stage_v[_srow, pl.ds(off, LANES)] = jnp.where(
                fin > 0.0, 1.0, -1.0)
            b_acc = lax.bitcast_convert_type(acc, jnp.int32)
            b_np = lax.bitcast_convert_type(-pre, jnp.int32)
            blend = b_acc ^ ((b_acc ^ b_np) & _hmask)
            acc_v[pl.ds(off, LANES)] = total + lax.bitcast_convert_type(
                blend, jnp.float32)

        @pl.when(hci > 0)
        def _():
            pltpu.make_async_copy(
                stage_v.at[srow], out_hbm.at[base + b0], sem_o).start()

        return b0 + hci, p + SEQ * hci - CHUNK

    def outer_body(i, carry):
        b0, p = carry
        for j in range(NBUF):
            kk = i * NBUF + j
            nslot = (j + 2) % NBUF
            pltpu.make_async_copy(
                table_hbm.at[idx_v.at[pl.ds(CHUNK * (kk + 2), CHUNK)]],
                bufs[nslot], sems[nslot]).start()
            b0, p = do_chunk(bufs[j], sems[j], b0, p)
        return b0, p

    b0, p = lax.fori_loop(0, NMAIN // NBUF, outer_body,
                          (jnp.int32(0), jnp.int32(SEQ)))

    b0, p = do_chunk(bufs[0], sems[0], b0, p)
    b0, p = do_chunk(bufs[1], sems[1], b0, p)

    for half in range(2):
        pltpu.make_async_copy(
            stage_v, out_hbm.at[pl.ds(base, NSTAGE)], sem_o).wait()


@jax.jit
def kernel(x, symbol):
    mesh = plsc.VectorSubcoreMesh(core_axis_name="c", subcore_axis_name="s")
    f = pl.kernel(
        _sc_encode,
        mesh=mesh,
        out_type=jax.ShapeDtypeStruct((BATCH, DIM), jnp.float32),
        scratch_types=[
            pltpu.VMEM((IDX_PER_W,), jnp.int32),
            pltpu.VMEM((CHUNK, DIM), jnp.float32),
            pltpu.VMEM((CHUNK, DIM), jnp.float32),
            pltpu.VMEM((CHUNK, DIM), jnp.float32),
            pltpu.VMEM((DIM,), jnp.float32),
            pltpu.VMEM((NSTAGE, DIM), jnp.float32),
            pltpu.SemaphoreType.DMA,
            pltpu.SemaphoreType.DMA,
            pltpu.SemaphoreType.DMA,
            pltpu.SemaphoreType.DMA,
        ],
    )
    return f(x.reshape(NUM_WORKERS, IDX_PER_W), symbol)


# speedup vs baseline: 1.1893x; 1.1485x over previous
import jax
import jax.numpy as jnp
from jax import lax
from jax.experimental import pallas as pl
from jax.experimental.pallas import tpu as pltpu
from jax.experimental.pallas import tpu_sc as plsc

BATCH = 1024
SEQ = 50
DIM = 1024
LANES = 16
NUM_CORES = 2
NUM_SUBCORES = 16
NUM_WORKERS = NUM_CORES * NUM_SUBCORES
BPW = BATCH // NUM_WORKERS
IDX_PER_W = BPW * SEQ
CHUNK = 32
NCH = IDX_PER_W // CHUNK
NBUF = 3
NMAIN = NCH - 2
NSTAGE = 16


def _sc_encode(x_hbm, table_hbm, out_hbm, idx_v, rows_a, rows_b, rows_c,
               acc_v, stage_v, sem_a, sem_b, sem_c, sem_o):
    wid = lax.axis_index("s") * NUM_CORES + lax.axis_index("c")
    base = wid * BPW
    pltpu.sync_copy(x_hbm.at[wid], idx_v)

    bufs = (rows_a, rows_b, rows_c)
    sems = (sem_a, sem_b, sem_c)

    for k in range(2):
        pltpu.make_async_copy(
            table_hbm.at[idx_v.at[pl.ds(CHUNK * k, CHUNK)]], bufs[k],
            sems[k]).start()

    def zero_body(c, carry):
        acc_v[pl.ds(c * LANES, LANES)] = jnp.zeros((LANES,), jnp.float32)
        return carry

    lax.fori_loop(0, DIM // LANES, zero_body, 0)

    def do_chunk(buf, sem, b0, p):
        pltpu.make_async_copy(table_hbm.at[pl.ds(0, CHUNK)], buf, sem).wait()

        hci = 1 + ((CHUNK - p) >> 31)
        srow = b0 & (NSTAGE - 1)
        row_masks = [
            jnp.full((LANES,), (r - p) >> 31, jnp.int32)
            for r in range(CHUNK)
        ]
        hmask = jnp.full((LANES,), -hci, jnp.int32)

        @plsc.parallel_loop(0, DIM // LANES, unroll=4)
        def c_body(c, _buf=buf, _masks=row_masks, _hmask=hmask, _srow=srow):
            off = c * LANES
            zero = jnp.zeros((LANES,), jnp.float32)
            t0 = t1 = t2 = t3 = zero
            a0 = a1c = zero
            for r in range(CHUNK):
                x = _buf[r, pl.ds(off, LANES)]
                if r % 4 == 0:
                    t0 = t0 + x
                elif r % 4 == 1:
                    t1 = t1 + x
                elif r % 4 == 2:
                    t2 = t2 + x
                else:
                    t3 = t3 + x
                xm = lax.bitcast_convert_type(
                    lax.bitcast_convert_type(x, jnp.int32) & _masks[r],
                    jnp.float32)
                if r % 2 == 0:
                    a0 = a0 + xm
                else:
                    a1c = a1c + xm
            total = (t0 + t1) + (t2 + t3)
            pre = a0 + a1c
            acc = acc_v[pl.ds(off, LANES)]
            fin = acc + pre
            stage_v[_srow, pl.ds(off, LANES)] = jnp.where(
                fin > 0.0, 1.0, -1.0)
            b_acc = lax.bitcast_convert_type(acc, jnp.int32)
            b_np = lax.bitcast_convert_type(-pre, jnp.int32)
            blend = b_acc ^ ((b_acc ^ b_np) & _hmask)
            acc_v[pl.ds(off, LANES)] = total + lax.bitcast_convert_type(
                blend, jnp.float32)

        @pl.when(hci > 0)
        def _():
            pltpu.make_async_copy(
                stage_v.at[srow], out_hbm.at[base + b0], sem_o).start()

        return b0 + hci, p + SEQ * hci - CHUNK

    def outer_body(i, carry):
        b0, p = carry
        for j in range(NBUF):
            kk = i * NBUF + j
            nslot = (j + 2) % NBUF
            pltpu.make_async_copy(
                table_hbm.at[idx_v.at[pl.ds(CHUNK * (kk + 2), CHUNK)]],
                bufs[nslot], sems[nslot]).start()
            b0, p = do_chunk(bufs[j], sems[j], b0, p)
        return b0, p

    b0, p = lax.fori_loop(0, NMAIN // NBUF, outer_body,
                          (jnp.int32(0), jnp.int32(SEQ)))

    b0, p = do_chunk(bufs[0], sems[0], b0, p)
    b0, p = do_chunk(bufs[1], sems[1], b0, p)

    for half in range(2):
        pltpu.make_async_copy(
            stage_v, out_hbm.at[pl.ds(base, NSTAGE)], sem_o).wait()


@jax.jit
def kernel(x, symbol):
    mesh = plsc.VectorSubcoreMesh(core_axis_name="c", subcore_axis_name="s")
    f = pl.kernel(
        _sc_encode,
        mesh=mesh,
        out_type=jax.ShapeDtypeStruct((BATCH, DIM), jnp.float32),
        scratch_types=[
            pltpu.VMEM((IDX_PER_W,), jnp.int32),
            pltpu.VMEM((CHUNK, DIM), jnp.float32),
            pltpu.VMEM((CHUNK, DIM), jnp.float32),
            pltpu.VMEM((CHUNK, DIM), jnp.float32),
            pltpu.VMEM((DIM,), jnp.float32),
            pltpu.VMEM((NSTAGE, DIM), jnp.float32),
            pltpu.SemaphoreType.DMA,
            pltpu.SemaphoreType.DMA,
            pltpu.SemaphoreType.DMA,
            pltpu.SemaphoreType.DMA,
        ],
    )
    return f(x.reshape(NUM_WORKERS, IDX_PER_W), symbol)


# speedup vs baseline: 1.2409x; 1.0434x over previous
import jax
import jax.numpy as jnp
from jax import lax
from jax.experimental import pallas as pl
from jax.experimental.pallas import tpu as pltpu
from jax.experimental.pallas import tpu_sc as plsc

BATCH = 1024
SEQ = 50
DIM = 1024
LANES = 16
NUM_CORES = 2
NUM_SUBCORES = 16
NUM_WORKERS = NUM_CORES * NUM_SUBCORES
BPW = BATCH // NUM_WORKERS
IDX_PER_W = BPW * SEQ
CHUNK = 32
NCH = IDX_PER_W // CHUNK
NBUF = 3
NMAIN = NCH - 2
NSTAGE = 16


def _sc_encode(x_hbm, table_hbm, out_hbm, idx_v, rows_a, rows_b, rows_c,
               acc_v, stage_v, sem_a, sem_b, sem_c, sem_o):
    wid = lax.axis_index("s") * NUM_CORES + lax.axis_index("c")
    base = wid * BPW
    pltpu.sync_copy(x_hbm.at[wid], idx_v)

    bufs = (rows_a, rows_b, rows_c)
    sems = (sem_a, sem_b, sem_c)

    for k in range(2):
        pltpu.make_async_copy(
            table_hbm.at[idx_v.at[pl.ds(CHUNK * k, CHUNK)]], bufs[k],
            sems[k]).start()

    def zero_body(c, carry):
        acc_v[pl.ds(c * LANES, LANES)] = jnp.zeros((LANES,), jnp.float32)
        return carry

    lax.fori_loop(0, DIM // LANES, zero_body, 0)

    def do_chunk(buf, sem, b0, p):
        pltpu.make_async_copy(table_hbm.at[pl.ds(0, CHUNK)], buf, sem).wait()

        hci = 1 + ((CHUNK - p) >> 31)
        srow = b0 & (NSTAGE - 1)
        row_masks = [
            jnp.full((LANES,), (r - p) >> 31, jnp.int32)
            for r in range(CHUNK)
        ]
        hmask = jnp.full((LANES,), -hci, jnp.int32)

        @plsc.parallel_loop(0, DIM // LANES, unroll=8)
        def c_body(c, _buf=buf, _masks=row_masks, _hmask=hmask, _srow=srow):
            off = c * LANES
            zero = jnp.zeros((LANES,), jnp.float32)
            t0 = t1 = t2 = t3 = zero
            a0 = a1c = zero
            for r in range(CHUNK):
                x = _buf[r, pl.ds(off, LANES)]
                if r % 4 == 0:
                    t0 = t0 + x
                elif r % 4 == 1:
                    t1 = t1 + x
                elif r % 4 == 2:
                    t2 = t2 + x
                else:
                    t3 = t3 + x
                xm = lax.bitcast_convert_type(
                    lax.bitcast_convert_type(x, jnp.int32) & _masks[r],
                    jnp.float32)
                if r % 2 == 0:
                    a0 = a0 + xm
                else:
                    a1c = a1c + xm
            total = (t0 + t1) + (t2 + t3)
            pre = a0 + a1c
            acc = acc_v[pl.ds(off, LANES)]
            fin = acc + pre
            stage_v[_srow, pl.ds(off, LANES)] = jnp.where(
                fin > 0.0, 1.0, -1.0)
            b_acc = lax.bitcast_convert_type(acc, jnp.int32)
            b_np = lax.bitcast_convert_type(-pre, jnp.int32)
            blend = b_acc ^ ((b_acc ^ b_np) & _hmask)
            acc_v[pl.ds(off, LANES)] = total + lax.bitcast_convert_type(
                blend, jnp.float32)

        @pl.when(hci > 0)
        def _():
            pltpu.make_async_copy(
                stage_v.at[srow], out_hbm.at[base + b0], sem_o).start()

        return b0 + hci, p + SEQ * hci - CHUNK

    def outer_body(i, carry):
        b0, p = carry
        for j in range(NBUF):
            kk = i * NBUF + j
            nslot = (j + 2) % NBUF
            pltpu.make_async_copy(
                table_hbm.at[idx_v.at[pl.ds(CHUNK * (kk + 2), CHUNK)]],
                bufs[nslot], sems[nslot]).start()
            b0, p = do_chunk(bufs[j], sems[j], b0, p)
        return b0, p

    b0, p = lax.fori_loop(0, NMAIN // NBUF, outer_body,
                          (jnp.int32(0), jnp.int32(SEQ)))

    b0, p = do_chunk(bufs[0], sems[0], b0, p)
    b0, p = do_chunk(bufs[1], sems[1], b0, p)

    for half in range(2):
        pltpu.make_async_copy(
            stage_v, out_hbm.at[pl.ds(base, NSTAGE)], sem_o).wait()


@jax.jit
def kernel(x, symbol):
    mesh = plsc.VectorSubcoreMesh(core_axis_name="c", subcore_axis_name="s")
    f = pl.kernel(
        _sc_encode,
        mesh=mesh,
        out_type=jax.ShapeDtypeStruct((BATCH, DIM), jnp.float32),
        scratch_types=[
            pltpu.VMEM((IDX_PER_W,), jnp.int32),
            pltpu.VMEM((CHUNK, DIM), jnp.float32),
            pltpu.VMEM((CHUNK, DIM), jnp.float32),
            pltpu.VMEM((CHUNK, DIM), jnp.float32),
            pltpu.VMEM((DIM,), jnp.float32),
            pltpu.VMEM((NSTAGE, DIM), jnp.float32),
            pltpu.SemaphoreType.DMA,
            pltpu.SemaphoreType.DMA,
            pltpu.SemaphoreType.DMA,
            pltpu.SemaphoreType.DMA,
        ],
    )
    return f(x.reshape(NUM_WORKERS, IDX_PER_W), symbol)
